# baseline (speedup 1.0000x reference)
import jax
import jax.numpy as jnp
from jax import lax
from jax.experimental import pallas as pl

DEG = 5
NC = 128
OUT_X = 512
OUT_Y = 512
KLEN = NC + DEG + 1
KPAD = 256
NOFF = 2 * DEG - 1

_F32 = jnp.float32
_HI = lax.Precision.HIGHEST
_MID = lax.Precision.DEFAULT


def _norm_knots(kraw_row, kraw_col):
    kp_row = jnp.where(kraw_row < 0.0, jnp.float32(1e-4), kraw_row)
    kp_col = jnp.where(kraw_col < 0.0, jnp.float32(1e-4), kraw_col)
    ii = lax.broadcasted_iota(jnp.int32, (KPAD, KPAD), 0)
    jj = lax.broadcasted_iota(jnp.int32, (KPAD, KPAD), 1)
    cum_row = jnp.sum(jnp.where(ii <= jj, kp_col, 0.0), axis=0, keepdims=True)
    cum_col = jnp.sum(jnp.where(jj <= ii, kp_row, 0.0), axis=1, keepdims=True)
    first = kp_row[0:1, 0:1]
    total = cum_row[0:1, KLEN - 1:KLEN]
    scale = 1.0 / (total - first)
    return (cum_row - first) * scale, (cum_col - first) * scale


def _shift_mat(kv_row):
    zr = jnp.zeros((1, DEG - 1), _F32)
    pad_row = jnp.concatenate([zr, kv_row, zr], axis=1)
    rows = [lax.slice(pad_row, (0, j), (1, j + KPAD)) for j in range(NOFF)]
    return jnp.concatenate(rows, axis=0)


def _basis(ev, g):
    left = [None] * (DEG + 1)
    right = [None] * (DEG + 1)
    for j in range(1, DEG + 1):
        left[j] = ev - g[1 - j]
        right[j] = g[j - 1] - ev
    basis = [jnp.ones_like(ev)] + [None] * DEG
    for j in range(1, DEG + 1):
        saved = jnp.zeros_like(ev)
        for r in range(j):
            temp = basis[r] / (right[r + 1] + left[j - r])
            basis[r] = saved + right[r + 1] * temp
            saved = left[j - r] * temp
        basis[j] = saved
    return basis


def _axis_weights(kv_row, kv_col, n_out):
    tt = lax.broadcasted_iota(jnp.int32, (1, n_out), 1)
    ev = 1e-5 + tt.astype(_F32) * ((1.0 - 2e-5) / (n_out - 1))
    cnt = jnp.sum((kv_col <= ev).astype(jnp.int32), axis=0, keepdims=True)
    s = cnt - 1
    s = jnp.where(ev == kv_row[0:1, NC:NC + 1], NC - 1, s)
    s = jnp.clip(s, DEG, NC - 1)

    k_io = lax.broadcasted_iota(jnp.int32, (KPAD, n_out), 0)
    sel = (k_io == s).astype(_F32)
    gm = jnp.dot(_shift_mat(kv_row), sel,
                 preferred_element_type=_F32, precision=_HI)
    g = {o: gm[o + DEG - 1:o + DEG, :] for o in range(-DEG + 1, DEG)}
    b = _basis(ev, g)

    c_io = lax.broadcasted_iota(jnp.int32, (NC, n_out), 0)
    base = s - DEG
    w = jnp.zeros((NC, n_out), _F32)
    for i in range(DEG + 1):
        w = jnp.where(c_io == base + i, b[i], w)
    return w


def _nurbs_kernel(pt_ref, kxr_ref, kxc_ref, kyr_ref, kyc_ref, out_ref):
    kvx_row, kvx_col = _norm_knots(kxr_ref[...], kxc_ref[...])
    kvy_row, kvy_col = _norm_knots(kyr_ref[...], kyc_ref[...])

    wxt = _axis_weights(kvx_row, kvx_col, OUT_X)
    wyt = _axis_weights(kvy_row, kvy_col, OUT_Y)

    a2 = lax.dot_general(wxt, pt_ref[...],
                         dimension_numbers=(((0,), (0,)), ((), ())),
                         preferred_element_type=_F32, precision=_MID)

    for d in range(3):
        acc = jnp.dot(a2[:, d * NC:(d + 1) * NC], wyt,
                      preferred_element_type=_F32, precision=_MID)
        out_ref[:, d * OUT_Y:(d + 1) * OUT_Y] = acc


def _call(pt, kxr, kxc, kyr, kyc, interpret=False):
    return pl.pallas_call(
        _nurbs_kernel,
        out_shape=jax.ShapeDtypeStruct((OUT_X, 3 * OUT_Y), _F32),
        interpret=interpret,
    )(pt, kxr, kxc, kyr, kyc)


def kernel(control_points, knot_vector_x, knot_vector_y):
    pt = control_points.transpose(0, 2, 1).reshape(NC, 3 * NC)
    kxr = jnp.pad(knot_vector_x, ((0, 0), (0, KPAD - KLEN)))
    kyr = jnp.pad(knot_vector_y, ((0, 0), (0, KPAD - KLEN)))
    kxc = kxr.reshape(KPAD, 1)
    kyc = kyr.reshape(KPAD, 1)
    out = _call(pt, kxr, kxc, kyr, kyc)
    return out.reshape(OUT_X, 3, OUT_Y).transpose(0, 2, 1)


# speedup vs baseline: 1.5181x; 1.5181x over previous
import jax
import jax.numpy as jnp
from jax import lax
from jax.experimental import pallas as pl

DEG = 5
NC = 128
OUT_X = 512
OUT_Y = 512
KLEN = NC + DEG + 1
KPAD = 256
NOFF = 2 * DEG - 1

_F32 = jnp.float32
_HI = lax.Precision.HIGHEST
_MID = lax.Precision.DEFAULT


def _norm_knots(kraw_row, kraw_col):
    kp_row = jnp.where(kraw_row < 0.0, jnp.float32(1e-4), kraw_row)
    kp_col = jnp.where(kraw_col < 0.0, jnp.float32(1e-4), kraw_col)
    ii = lax.broadcasted_iota(jnp.int32, (KPAD, KPAD), 0)
    jj = lax.broadcasted_iota(jnp.int32, (KPAD, KPAD), 1)
    cum_row = jnp.sum(jnp.where(ii <= jj, kp_col, 0.0), axis=0, keepdims=True)
    cum_col = jnp.sum(jnp.where(jj <= ii, kp_row, 0.0), axis=1, keepdims=True)
    first = kp_row[0:1, 0:1]
    total = cum_row[0:1, KLEN - 1:KLEN]
    scale = 1.0 / (total - first)
    return (cum_row - first) * scale, (cum_col - first) * scale


def _shift_mat(kv_row):
    zr = jnp.zeros((1, DEG - 1), _F32)
    pad_row = jnp.concatenate([zr, kv_row, zr], axis=1)
    rows = [lax.slice(pad_row, (0, j), (1, j + KPAD)) for j in range(NOFF)]
    return jnp.concatenate(rows, axis=0)


def _basis(ev, g):
    left = [None] * (DEG + 1)
    right = [None] * (DEG + 1)
    for j in range(1, DEG + 1):
        left[j] = ev - g[1 - j]
        right[j] = g[j - 1] - ev
    basis = [jnp.ones_like(ev)] + [None] * DEG
    for j in range(1, DEG + 1):
        saved = jnp.zeros_like(ev)
        for r in range(j):
            temp = basis[r] / (right[r + 1] + left[j - r])
            basis[r] = saved + right[r + 1] * temp
            saved = left[j - r] * temp
        basis[j] = saved
    return basis


def _axis_weights(kv_row, kv_col, n_out):
    tt = lax.broadcasted_iota(jnp.int32, (1, n_out), 1)
    ev = 1e-5 + tt.astype(_F32) * ((1.0 - 2e-5) / (n_out - 1))
    cnt = jnp.sum((kv_col <= ev).astype(jnp.int32), axis=0, keepdims=True)
    s = cnt - 1
    s = jnp.where(ev == kv_row[0:1, NC:NC + 1], NC - 1, s)
    s = jnp.clip(s, DEG, NC - 1)

    k_io = lax.broadcasted_iota(jnp.int32, (KPAD, n_out), 0)
    sel = (k_io == s).astype(_F32)
    gm = jnp.dot(_shift_mat(kv_row), sel,
                 preferred_element_type=_F32, precision=_HI)
    g = {o: gm[o + DEG - 1:o + DEG, :] for o in range(-DEG + 1, DEG)}
    b = _basis(ev, g)

    c_io = lax.broadcasted_iota(jnp.int32, (NC, n_out), 0)
    base = s - DEG
    w = jnp.zeros((NC, n_out), _F32)
    for i in range(DEG + 1):
        w = jnp.where(c_io == base + i, b[i], w)
    return w


def _nurbs_kernel(pt_ref, kxr_ref, kxc_ref, kyr_ref, kyc_ref, out_ref):
    kvx_row, kvx_col = _norm_knots(kxr_ref[...], kxc_ref[...])
    kvy_row, kvy_col = _norm_knots(kyr_ref[...], kyc_ref[...])

    wxt = _axis_weights(kvx_row, kvx_col, OUT_X)
    wyt = _axis_weights(kvy_row, kvy_col, OUT_Y)

    a2 = lax.dot_general(wxt, pt_ref[...],
                         dimension_numbers=(((0,), (0,)), ((), ())),
                         preferred_element_type=_F32, precision=_MID)

    for d in range(3):
        acc = jnp.dot(a2[:, d * NC:(d + 1) * NC], wyt,
                      preferred_element_type=_F32, precision=_MID)
        out_ref[d * OUT_X:(d + 1) * OUT_X, :] = acc


def _call(pt, kxr, kxc, kyr, kyc, interpret=False):
    return pl.pallas_call(
        _nurbs_kernel,
        out_shape=jax.ShapeDtypeStruct((3 * OUT_X, OUT_Y), _F32),
        interpret=interpret,
    )(pt, kxr, kxc, kyr, kyc)


def kernel(control_points, knot_vector_x, knot_vector_y):
    pt = control_points.transpose(0, 2, 1).reshape(NC, 3 * NC)
    kxr = jnp.pad(knot_vector_x, ((0, 0), (0, KPAD - KLEN)))
    kyr = jnp.pad(knot_vector_y, ((0, 0), (0, KPAD - KLEN)))
    kxc = kxr.reshape(KPAD, 1)
    kyc = kyr.reshape(KPAD, 1)
    out = _call(pt, kxr, kxc, kyr, kyc)
    return out.reshape(3, OUT_X, OUT_Y).transpose(1, 2, 0)
